# baseline (speedup 1.0000x reference)
import functools

import jax
import jax.numpy as jnp
from jax import lax
from jax.experimental import pallas as pl
from jax.experimental.pallas import tpu as pltpu
from jax.experimental.pallas import tpu_sc as plsc

_B, _N, _E, _D, _EF, _P = 64, 128, 4096, 128, 16, 512
_NC, _NS = 2, 16
_NW = _NC * _NS
_PAIRS = _B * _P
_PW = _PAIRS // _NW
_IDXW = _PW * _EF
_RPD = 128
_NDMA = _IDXW // _RPD


def _edge_feature_gather(table_flat, pairs_lin):
    mesh = plsc.VectorSubcoreMesh(core_axis_name="c", subcore_axis_name="s")

    @functools.partial(
        pl.kernel,
        out_type=jax.ShapeDtypeStruct((_EF, _PAIRS), jnp.float32),
        mesh=mesh,
        scratch_types=[
            pltpu.VMEM((2 * _PW,), jnp.int32),
            pltpu.VMEM((_NDMA, _RPD), jnp.int32),
            pltpu.VMEM((_IDXW,), jnp.float32),
            pltpu.SemaphoreType.DMA,
        ],
        compiler_params=pltpu.CompilerParams(use_tc_tiling_on_sc=False,
                                             needs_layout_passes=False),
    )
    def gather_kernel(table_hbm, pairs_hbm, out_hbm, pq_v, idx_v, rows_v, sem):
        wid = lax.axis_index("s") * _NC + lax.axis_index("c")
        pltpu.sync_copy(pairs_hbm.at[pl.ds(wid * 2 * _PW, 2 * _PW)], pq_v)
        copies = []
        for w in range(8):
            for c8 in range(8):
                c = w * 8 + c8
                blo = (c // 32) * 1024 + ((c % 32) // 8) * 256 + (c % 8) * 16
                i0 = pq_v[pl.ds(blo, 16)]
                i1 = pq_v[pl.ds(blo + 128, 16)]
                b = wid * (_PW // _P) + c // 32
                a_c = (jnp.full((16,), b * (_N * _EF * _N), jnp.int32)
                       + i0 * (_EF * _N) + i1)

                def fbody(f, _):
                    idx_v[f * 8 + w, pl.ds(c8 * 16, 16)] = (
                        a_c + jnp.full((16,), f, jnp.int32) * _N)
                    return 0

                lax.fori_loop(0, _EF, fbody, 0)
            for f in range(_EF):
                k = f * 8 + w
                copies.append(
                    pltpu.async_copy(table_hbm.at[idx_v.at[k]],
                                     rows_v.at[pl.ds(k * _RPD, _RPD)], sem))
        for cp in copies:
            cp.wait()
        for f in range(_EF):
            pltpu.sync_copy(rows_v.at[pl.ds(f * _PW, _PW)],
                            out_hbm.at[f, pl.ds(wid * _PW, _PW)])

    return gather_kernel(table_flat, pairs_lin)


_BPS = 4


def _tc_body(x_ref, ei_ref, op_ref, ee_ref,
             wg1, bg1, wg2, bg2,
             wa_cat, wb_cat, b1_cat, w2_blk, b2_blk,
             lr_ref, cr_ref, mr_ref):
    for i in range(_BPS):
        src = ei_ref[i, 0:1, :]
        dst = ei_ref[i, 1:2, :]
        sub_iota = lax.broadcasted_iota(jnp.int32, (_N, _E), 0)
        st = (src == sub_iota).astype(jnp.bfloat16)
        dt = (dst == sub_iota).astype(jnp.bfloat16)
        a = lax.dot_general(dt, st, (((1,), (1,)), ((), ())),
                            preferred_element_type=jnp.float32)
        deg = jnp.maximum(
            jnp.dot(dt, jnp.ones((_E, 1), jnp.bfloat16),
                    preferred_element_type=jnp.float32), 1.0)

        h = x_ref[i]
        m = jnp.dot(a, h, preferred_element_type=jnp.float32) / deg
        h = jnp.maximum(
            jnp.dot(m, wg1[...], preferred_element_type=jnp.float32)
            + bg1[...], 0.0)
        m = jnp.dot(a, h, preferred_element_type=jnp.float32) / deg
        h = jnp.maximum(
            jnp.dot(m, wg2[...], preferred_element_type=jnp.float32)
            + bg2[...], 0.0)

        pt = jnp.transpose(op_ref[i])
        lane_n = lax.broadcasted_iota(jnp.int32, (_N, _N), 1)
        ohp = jnp.concatenate(
            [(pt[:, 2 * blk:2 * blk + 1] == lane_n).astype(jnp.float32)
             + (pt[:, 2 * blk + 1:2 * blk + 2] == lane_n).astype(jnp.float32)
             for blk in range(_P // _N)], axis=0)
        emb = jnp.dot(ohp, h, preferred_element_type=jnp.float32)
        eet = ee_ref[:, i, 0, :]

        hh = jnp.maximum(
            jnp.dot(emb, wa_cat[...], preferred_element_type=jnp.float32)
            + lax.dot_general(eet, wb_cat[...], (((0,), (0,)), ((), ())),
                              preferred_element_type=jnp.float32)
            + b1_cat[...], 0.0)
        hall = (lax.dot_general(w2_blk[...], hh, (((0,), (1,)), ((), ())),
                                preferred_element_type=jnp.float32)
                + b2_blk[...])
        lr_ref[:, i, 0, :] = hall[0:10]
        cr_ref[:, i, 0, :] = hall[16:42]
        mr_ref[:, i, 0, :] = hall[48:53]


def _full(shape):
    return pl.BlockSpec(shape, lambda b: (0,) * len(shape))


def kernel(concatenated_node_features, edge_index, interaction_feature,
           object_pairs,
           W_g1, b_g1, W_g2, b_g2,
           W_cr1, b_cr1, W_cr2, b_cr2,
           W_lr1, b_lr1, W_lr2, b_lr2,
           W_mr1, b_mr1, W_mr2, b_mr2):
    x = concatenated_node_features

    table_flat = interaction_feature.transpose(0, 1, 3, 2).reshape(-1)
    pairs_lin = (object_pairs.reshape(_B, _P // _N, _N, 2)
                 .transpose(0, 1, 3, 2).reshape(-1))
    eet = _edge_feature_gather(table_flat, pairs_lin).reshape(_EF, _B, 1, _P)

    wa_cat = jnp.concatenate([W_lr1[:_D], W_cr1[:_D], W_mr1[:_D]], axis=1)
    wb_cat = jnp.concatenate([W_lr1[_D:], W_cr1[_D:], W_mr1[_D:]], axis=1)
    b1_cat = jnp.concatenate([b_lr1, b_cr1, b_mr1]).reshape(1, -1)
    w2_blk = jnp.zeros((192, 56), jnp.float32)
    w2_blk = w2_blk.at[0:64, 0:10].set(W_lr2)
    w2_blk = w2_blk.at[64:128, 16:42].set(W_cr2)
    w2_blk = w2_blk.at[128:192, 48:53].set(W_mr2)
    b2_blk = jnp.zeros((56, 1), jnp.float32)
    b2_blk = b2_blk.at[0:10, 0].set(b_lr2)
    b2_blk = b2_blk.at[16:42, 0].set(b_cr2)
    b2_blk = b2_blk.at[48:53, 0].set(b_mr2)

    out_shapes = [jax.ShapeDtypeStruct((c, _B, 1, _P), jnp.float32)
                  for c in (10, 26, 5)]
    per_b = lambda shape: pl.BlockSpec(shape, lambda b: (b, 0, 0))

    lrt, crt, mrt = pl.pallas_call(
        _tc_body,
        grid=(_B // _BPS,),
        in_specs=[
            per_b((_BPS, _N, _D)),
            per_b((_BPS, 2, _E)),
            per_b((_BPS, 2 * _P // _N, _N)),
            pl.BlockSpec((_EF, _BPS, 1, _P), lambda b: (0, b, 0, 0)),
            _full((_D, _D)), _full((1, _D)), _full((_D, _D)), _full((1, _D)),
            _full((_D, 192)), _full((_EF, 192)), _full((1, 192)),
            _full((192, 56)), _full((56, 1)),
        ],
        out_specs=[pl.BlockSpec((10, _BPS, 1, _P), lambda b: (0, b, 0, 0)),
                   pl.BlockSpec((26, _BPS, 1, _P), lambda b: (0, b, 0, 0)),
                   pl.BlockSpec((5, _BPS, 1, _P), lambda b: (0, b, 0, 0))],
        out_shape=out_shapes,
    )(x, edge_index, pairs_lin.reshape(_B, 2 * _P // _N, _N), eet,
      W_g1, b_g1.reshape(1, -1), W_g2, b_g2.reshape(1, -1),
      wa_cat, wb_cat, b1_cat, w2_blk, b2_blk)
    lr = lrt.reshape(10, _B, _P).transpose(1, 2, 0)
    cr = crt.reshape(26, _B, _P).transpose(1, 2, 0)
    mr = mrt.reshape(5, _B, _P).transpose(1, 2, 0)
    return (lr, cr, mr)


# speedup vs baseline: 1.1541x; 1.1541x over previous
import functools

import jax
import jax.numpy as jnp
from jax import lax
from jax.experimental import pallas as pl
from jax.experimental.pallas import tpu as pltpu
from jax.experimental.pallas import tpu_sc as plsc

_B, _N, _E, _D, _EF, _P = 64, 128, 4096, 128, 16, 512
_NC, _NS = 2, 16
_NW = _NC * _NS
_PAIRS = _B * _P
_PW = _PAIRS // _NW
_IDXW = _PW * _EF
_RPD = 128
_NDMA = _IDXW // _RPD


def _edge_feature_gather(table_flat, pairs_lin):
    mesh = plsc.VectorSubcoreMesh(core_axis_name="c", subcore_axis_name="s")

    @functools.partial(
        pl.kernel,
        out_type=jax.ShapeDtypeStruct((_EF, _PAIRS), jnp.float32),
        mesh=mesh,
        scratch_types=[
            pltpu.VMEM((2 * _PW,), jnp.int32),
            pltpu.VMEM((_NDMA, _RPD), jnp.int32),
            pltpu.VMEM((_IDXW,), jnp.float32),
            pltpu.SemaphoreType.DMA,
        ],
        compiler_params=pltpu.CompilerParams(use_tc_tiling_on_sc=False,
                                             needs_layout_passes=False),
    )
    def gather_kernel(table_hbm, pairs_hbm, out_hbm, pq_v, idx_v, rows_v, sem):
        wid = lax.axis_index("s") * _NC + lax.axis_index("c")
        pltpu.sync_copy(pairs_hbm.at[pl.ds(wid * 2 * _PW, 2 * _PW)], pq_v)
        copies = []
        for w in range(8):
            for c8 in range(8):
                c = w * 8 + c8
                blo = (c // 32) * 1024 + ((c % 32) // 8) * 256 + (c % 8) * 16
                i0 = pq_v[pl.ds(blo, 16)]
                i1 = pq_v[pl.ds(blo + 128, 16)]
                b = wid * (_PW // _P) + c // 32
                a_c = (jnp.full((16,), b * (_N * _EF * _N), jnp.int32)
                       + i0 * (_EF * _N) + i1)

                def fbody(f, _):
                    idx_v[f * 8 + w, pl.ds(c8 * 16, 16)] = (
                        a_c + jnp.full((16,), f, jnp.int32) * _N)
                    return 0

                lax.fori_loop(0, _EF, fbody, 0)
            for f in range(_EF):
                k = f * 8 + w
                copies.append(
                    pltpu.async_copy(table_hbm.at[idx_v.at[k]],
                                     rows_v.at[pl.ds(k * _RPD, _RPD)], sem))
        for cp in copies:
            cp.wait()
        for f in range(_EF):
            pltpu.sync_copy(rows_v.at[pl.ds(f * _PW, _PW)],
                            out_hbm.at[f, pl.ds(wid * _PW, _PW)])

    return gather_kernel(table_flat, pairs_lin)


_BPS = 4


def _gcn_body(x_ref, ei_ref, wg1, bg1, wg2, bg2, h2_ref):
    for i in range(_BPS):
        src = ei_ref[i, 0:1, :]
        dst = ei_ref[i, 1:2, :]
        sub_iota = lax.broadcasted_iota(jnp.int32, (_N, _E), 0)
        st = (src == sub_iota).astype(jnp.bfloat16)
        dt = (dst == sub_iota).astype(jnp.bfloat16)
        a = lax.dot_general(dt, st, (((1,), (1,)), ((), ())),
                            preferred_element_type=jnp.float32)
        deg = jnp.maximum(
            jnp.dot(dt, jnp.ones((_E, 1), jnp.bfloat16),
                    preferred_element_type=jnp.float32), 1.0)

        h = x_ref[i]
        m = jnp.dot(a, h, preferred_element_type=jnp.float32) / deg
        h = jnp.maximum(
            jnp.dot(m, wg1[...], preferred_element_type=jnp.float32)
            + bg1[...], 0.0)
        m = jnp.dot(a, h, preferred_element_type=jnp.float32) / deg
        h2_ref[i] = jnp.maximum(
            jnp.dot(m, wg2[...], preferred_element_type=jnp.float32)
            + bg2[...], 0.0)


def _head_body(h2_ref, op_ref, ee_ref,
               wa_cat, wb_cat, b1_cat, w2_blk, b2_blk,
               lr_ref, cr_ref, mr_ref):
    for i in range(_BPS):
        h = h2_ref[i]
        pt = jnp.transpose(op_ref[i])
        lane_n = lax.broadcasted_iota(jnp.int32, (_N, _N), 1)
        ohp = jnp.concatenate(
            [(pt[:, 2 * blk:2 * blk + 1] == lane_n).astype(jnp.float32)
             + (pt[:, 2 * blk + 1:2 * blk + 2] == lane_n).astype(jnp.float32)
             for blk in range(_P // _N)], axis=0)
        emb = jnp.dot(ohp, h, preferred_element_type=jnp.float32)
        eet = ee_ref[:, i, 0, :]

        hh = jnp.maximum(
            jnp.dot(emb, wa_cat[...], preferred_element_type=jnp.float32)
            + lax.dot_general(eet, wb_cat[...], (((0,), (0,)), ((), ())),
                              preferred_element_type=jnp.float32)
            + b1_cat[...], 0.0)
        hall = (lax.dot_general(w2_blk[...], hh, (((0,), (1,)), ((), ())),
                                preferred_element_type=jnp.float32)
                + b2_blk[...])
        lr_ref[:, i, 0, :] = hall[0:10]
        cr_ref[:, i, 0, :] = hall[16:42]
        mr_ref[:, i, 0, :] = hall[48:53]


def _full(shape):
    return pl.BlockSpec(shape, lambda b: (0,) * len(shape))


def kernel(concatenated_node_features, edge_index, interaction_feature,
           object_pairs,
           W_g1, b_g1, W_g2, b_g2,
           W_cr1, b_cr1, W_cr2, b_cr2,
           W_lr1, b_lr1, W_lr2, b_lr2,
           W_mr1, b_mr1, W_mr2, b_mr2):
    x = concatenated_node_features

    table_flat = interaction_feature.transpose(0, 1, 3, 2).reshape(-1)
    pairs_lin = (object_pairs.reshape(_B, _P // _N, _N, 2)
                 .transpose(0, 1, 3, 2).reshape(-1))
    eet = _edge_feature_gather(table_flat, pairs_lin).reshape(_EF, _B, 1, _P)

    wa_cat = jnp.concatenate([W_lr1[:_D], W_cr1[:_D], W_mr1[:_D]], axis=1)
    wb_cat = jnp.concatenate([W_lr1[_D:], W_cr1[_D:], W_mr1[_D:]], axis=1)
    b1_cat = jnp.concatenate([b_lr1, b_cr1, b_mr1]).reshape(1, -1)
    w2_blk = jnp.zeros((192, 56), jnp.float32)
    w2_blk = w2_blk.at[0:64, 0:10].set(W_lr2)
    w2_blk = w2_blk.at[64:128, 16:42].set(W_cr2)
    w2_blk = w2_blk.at[128:192, 48:53].set(W_mr2)
    b2_blk = jnp.zeros((56, 1), jnp.float32)
    b2_blk = b2_blk.at[0:10, 0].set(b_lr2)
    b2_blk = b2_blk.at[16:42, 0].set(b_cr2)
    b2_blk = b2_blk.at[48:53, 0].set(b_mr2)

    out_shapes = [jax.ShapeDtypeStruct((c, _B, 1, _P), jnp.float32)
                  for c in (10, 26, 5)]
    per_b = lambda shape: pl.BlockSpec(shape, lambda b: (b, 0, 0))

    h2 = pl.pallas_call(
        _gcn_body,
        grid=(_B // _BPS,),
        in_specs=[
            per_b((_BPS, _N, _D)),
            per_b((_BPS, 2, _E)),
            _full((_D, _D)), _full((1, _D)), _full((_D, _D)), _full((1, _D)),
        ],
        out_specs=per_b((_BPS, _N, _D)),
        out_shape=jax.ShapeDtypeStruct((_B, _N, _D), jnp.float32),
    )(x, edge_index, W_g1, b_g1.reshape(1, -1), W_g2, b_g2.reshape(1, -1))

    lrt, crt, mrt = pl.pallas_call(
        _head_body,
        grid=(_B // _BPS,),
        in_specs=[
            per_b((_BPS, _N, _D)),
            per_b((_BPS, 2 * _P // _N, _N)),
            pl.BlockSpec((_EF, _BPS, 1, _P), lambda b: (0, b, 0, 0)),
            _full((_D, 192)), _full((_EF, 192)), _full((1, 192)),
            _full((192, 56)), _full((56, 1)),
        ],
        out_specs=[pl.BlockSpec((10, _BPS, 1, _P), lambda b: (0, b, 0, 0)),
                   pl.BlockSpec((26, _BPS, 1, _P), lambda b: (0, b, 0, 0)),
                   pl.BlockSpec((5, _BPS, 1, _P), lambda b: (0, b, 0, 0))],
        out_shape=out_shapes,
    )(h2, pairs_lin.reshape(_B, 2 * _P // _N, _N), eet,
      wa_cat, wb_cat, b1_cat, w2_blk, b2_blk)
    lr = lrt.reshape(10, _B, _P).transpose(1, 2, 0)
    cr = crt.reshape(26, _B, _P).transpose(1, 2, 0)
    mr = mrt.reshape(5, _B, _P).transpose(1, 2, 0)
    return (lr, cr, mr)
